# baseline (speedup 1.0000x reference)
import functools
import jax
import jax.numpy as jnp
from jax import lax
from jax.experimental import pallas as pl
from jax.experimental.pallas import tpu as pltpu
from jax.experimental.pallas import tpu_sc as plsc

N_NODE = 10000
D = 128
E = 160000

NS = 16
CHUNK = 128
SUPER = 8
CHUNKS_PER_TILE = 160
SUPERS_PER_TILE = CHUNKS_PER_TILE // SUPER
EDGES_PER_TILE = CHUNK * CHUNKS_PER_TILE
E_PAD = EDGES_PER_TILE * NS
IDX_ROWS = NS * CHUNKS_PER_TILE
ROWS_PER_TILE = 640
N_ACC = ROWS_PER_TILE * NS
ROW_LOOPS = ROWS_PER_TILE // CHUNK


def _sc_aggregate(x_user, x_item, src_ui, dst_ui, src_iu, dst_iu,
                  ones_vec, z_vec, z_rows):
  mesh = plsc.VectorSubcoreMesh(core_axis_name="c", subcore_axis_name="s")

  @functools.partial(
      pl.kernel,
      out_type=(
          jax.ShapeDtypeStruct((N_ACC, D), jnp.float32),
          jax.ShapeDtypeStruct((N_ACC,), jnp.float32),
          jax.ShapeDtypeStruct((N_ACC, D), jnp.float32),
          jax.ShapeDtypeStruct((N_ACC,), jnp.float32),
      ),
      mesh=mesh,
      scratch_types=[
          pltpu.VMEM_SHARED((N_ACC, D), jnp.float32),
          pltpu.VMEM_SHARED((N_ACC,), jnp.float32),
          pltpu.VMEM((SUPER, CHUNK), jnp.int32),
          pltpu.VMEM((SUPER, CHUNK), jnp.int32),
          pltpu.VMEM((CHUNK, D), jnp.float32),
          pltpu.VMEM((CHUNK, D), jnp.float32),
          pltpu.VMEM((CHUNK,), jnp.float32),
          pltpu.SemaphoreType.DMA,
          pltpu.SemaphoreType.DMA,
          pltpu.SemaphoreType.DMA,
          pltpu.SemaphoreType.DMA,
          pltpu.SemaphoreType.DMA,
          pltpu.SemaphoreType.DMA,
          pltpu.SemaphoreType.DMA,
          pltpu.SemaphoreType.DMA,
          pltpu.SemaphoreType.DMA,
      ],
  )
  def k(x_u, x_i, s_ui, d_ui, s_iu, d_iu, ones_h, zv_h, z_h,
        agg_i_out, cnt_i_out, agg_u_out, cnt_u_out,
        acc_sh, cnt_sh, idx_s, idx_d, rows0, rows1, ones_v,
        sem0, sem1, sem2, sem3, ssem0, ssem1, ssem2, ssem3, semc):
    c = lax.axis_index("c")
    s = lax.axis_index("s")
    rbase = s * ROWS_PER_TILE
    bufs = (rows0, rows1)
    sems = (sem0, sem1)
    ssems = (ssem0, ssem1)

    rslice = pl.ds(rbase, ROWS_PER_TILE)
    pltpu.sync_copy(z_h.at[rslice], acc_sh.at[rslice])
    pltpu.sync_copy(zv_h.at[rslice], cnt_sh.at[rslice])
    pltpu.sync_copy(ones_h, ones_v)
    plsc.subcore_barrier()

    def run_edges(x_src, src2_h, dst2_h):
      irow = s * CHUNKS_PER_TILE

      def body(g, carry):
        pltpu.sync_copy(src2_h.at[pl.ds(irow + g * SUPER, SUPER)], idx_s)
        pltpu.sync_copy(dst2_h.at[pl.ds(irow + g * SUPER, SUPER)], idx_d)
        cnt_descs = []
        NB = len(bufs)
        descs = [pltpu.async_copy(x_src.at[idx_s.at[j]], bufs[j], sems[j])
                 for j in range(NB - 1)]
        descs.append(None)
        scat = [None] * NB
        for j in range(SUPER):
          b = j % NB
          descs[b].wait()
          nxt = j + NB - 1
          if nxt < SUPER:
            nb = nxt % NB
            if scat[nb] is not None:
              scat[nb].wait()
              scat[nb] = None
            descs[nb] = pltpu.async_copy(
                x_src.at[idx_s.at[nxt]], bufs[nb], sems[nb])
          scat[b] = pltpu.async_copy(bufs[b], acc_sh.at[idx_d.at[j]],
                                     ssems[b], add=True)
          cnt_descs.append(
              pltpu.async_copy(ones_v, cnt_sh.at[idx_d.at[j]], semc,
                               add=True))
        for dsc in scat:
          if dsc is not None:
            dsc.wait()
        for dsc in cnt_descs:
          dsc.wait()
        return carry

      lax.fori_loop(0, SUPERS_PER_TILE, body, 0)

    @pl.when(c == 0)
    def _():
      run_edges(x_u, s_ui, d_ui)

    @pl.when(c == 1)
    def _():
      run_edges(x_i, s_iu, d_iu)

    plsc.subcore_barrier()

    def writeout(agg_out, cnt_out):
      pltpu.sync_copy(acc_sh.at[rslice], agg_out.at[rslice])
      pltpu.sync_copy(cnt_sh.at[rslice], cnt_out.at[rslice])

    @pl.when(c == 0)
    def _():
      writeout(agg_i_out, cnt_i_out)

    @pl.when(c == 1)
    def _():
      writeout(agg_u_out, cnt_u_out)

  return k(x_user, x_item, src_ui, dst_ui, src_iu, dst_iu,
           ones_vec, z_vec, z_rows)


def _tc_post_one(agg, cnt, x_dst, W_l, b_l, W_r, alpha, ln_w, ln_b):
  def body(agg_r, cnt_r, x_r, wl_r, bl_r, wr_r, alpha_r, lnw_r, lnb_r,
           out_r):
    a = alpha_r[0, 0]
    inv_n = 1.0 / (N_NODE * D)
    c = jnp.maximum(cnt_r[...][:N_NODE], 1.0)
    agg_m = agg_r[...][:N_NODE] / c
    h = lax.dot_general(agg_m, wl_r[...], (((1,), (1,)), ((), ())),
                        precision=lax.Precision.HIGHEST,
                        preferred_element_type=jnp.float32)
    h = h + bl_r[...]
    h = h + lax.dot_general(x_r[...], wr_r[...], (((1,), (1,)), ((), ())),
                            precision=lax.Precision.HIGHEST,
                            preferred_element_type=jnp.float32)
    p = jnp.where(h >= 0.0, h, a * h)
    m = jnp.sum(p) * inv_n
    xc = p - m
    v = jnp.sum(xc * xc) * inv_n
    out_r[...] = xc * lax.rsqrt(v + 1e-5) * lnw_r[...] + lnb_r[...]

  return pl.pallas_call(
      body,
      out_shape=jax.ShapeDtypeStruct((N_NODE, D), jnp.float32),
  )(agg, cnt, x_dst, W_l, b_l.reshape(1, D), W_r,
    alpha.reshape(1, 1), ln_w.reshape(1, D), ln_b.reshape(1, D))


@jax.jit
def kernel(x_user, x_item, edge_index_ui, edge_index_iu, batch_user,
           batch_item, batch_size, W_l_ui, b_l_ui, W_r_ui, W_l_iu, b_l_iu,
           W_r_iu, alpha, ln_w_user, ln_b_user, ln_w_item, ln_b_item):
  del batch_user, batch_item

  pad = E_PAD - E
  pad_src = jnp.zeros((pad,), jnp.int32)
  pad_dst = jnp.full((pad,), N_NODE, jnp.int32)

  def prep(v, p):
    return jnp.concatenate([v, p]).reshape(IDX_ROWS, CHUNK)

  src_ui = prep(edge_index_ui[0], pad_src)
  dst_ui = prep(edge_index_ui[1], pad_dst)
  src_iu = prep(edge_index_iu[0], pad_src)
  dst_iu = prep(edge_index_iu[1], pad_dst)

  ones_vec = jnp.ones((CHUNK,), jnp.float32)
  z_vec = jnp.zeros((N_ACC,), jnp.float32)
  z_rows = jnp.zeros((N_ACC, D), jnp.float32)

  agg_i, cnt_i, agg_u, cnt_u = _sc_aggregate(
      x_user, x_item, src_ui, dst_ui, src_iu, dst_iu, ones_vec, z_vec, z_rows)

  out_user = _tc_post_one(agg_u, cnt_u.reshape(N_ACC, 1),
                          x_user, W_l_iu, b_l_iu, W_r_iu,
                          alpha, ln_w_user, ln_b_user)
  out_item = _tc_post_one(agg_i, cnt_i.reshape(N_ACC, 1),
                          x_item, W_l_ui, b_l_ui, W_r_ui,
                          alpha, ln_w_item, ln_b_item)
  return out_user, out_item


# speedup vs baseline: 16.4137x; 16.4137x over previous
import functools
import jax
import jax.numpy as jnp
from jax import lax
from jax.experimental import pallas as pl
from jax.experimental.pallas import tpu as pltpu
from jax.experimental.pallas import tpu_sc as plsc

N_NODE = 10000
D = 128
E = 160000

NS = 16
CHUNK = 32
SUPER = 32
CHUNKS_PER_TILE = 320
SUPERS_PER_TILE = CHUNKS_PER_TILE // SUPER
EDGES_PER_TILE = CHUNK * CHUNKS_PER_TILE
E_PAD = EDGES_PER_TILE * NS
IDX_ROWS = NS * CHUNKS_PER_TILE
ROWS_PER_TILE = 640
N_ACC = ROWS_PER_TILE * NS
ROW_LOOPS = ROWS_PER_TILE // CHUNK


def _sc_aggregate(x_user, x_item, src_ui, dst_ui, src_iu, dst_iu,
                  ones_vec, z_vec, z_rows):
  mesh = plsc.VectorSubcoreMesh(core_axis_name="c", subcore_axis_name="s")

  @functools.partial(
      pl.kernel,
      out_type=(
          jax.ShapeDtypeStruct((N_ACC, D), jnp.float32),
          jax.ShapeDtypeStruct((N_ACC,), jnp.float32),
          jax.ShapeDtypeStruct((N_ACC, D), jnp.float32),
          jax.ShapeDtypeStruct((N_ACC,), jnp.float32),
      ),
      mesh=mesh,
      scratch_types=[
          pltpu.VMEM_SHARED((N_ACC, D), jnp.float32),
          pltpu.VMEM_SHARED((N_ACC,), jnp.float32),
          pltpu.VMEM((SUPER, CHUNK), jnp.int32),
          pltpu.VMEM((SUPER, CHUNK), jnp.int32),
          pltpu.VMEM((CHUNK, D), jnp.float32),
          pltpu.VMEM((CHUNK, D), jnp.float32),
          pltpu.VMEM((CHUNK, D), jnp.float32),
          pltpu.VMEM((CHUNK, D), jnp.float32),
          pltpu.VMEM((CHUNK,), jnp.float32),
          pltpu.SemaphoreType.DMA,
          pltpu.SemaphoreType.DMA,
          pltpu.SemaphoreType.DMA,
          pltpu.SemaphoreType.DMA,
          pltpu.SemaphoreType.DMA,
          pltpu.SemaphoreType.DMA,
          pltpu.SemaphoreType.DMA,
          pltpu.SemaphoreType.DMA,
          pltpu.SemaphoreType.DMA,
      ],
  )
  def k(x_u, x_i, s_ui, d_ui, s_iu, d_iu, ones_h, zv_h, z_h,
        agg_i_out, cnt_i_out, agg_u_out, cnt_u_out,
        acc_sh, cnt_sh, idx_s, idx_d, rows0, rows1, rows2, rows3, ones_v,
        sem0, sem1, sem2, sem3, ssem0, ssem1, ssem2, ssem3, semc):
    c = lax.axis_index("c")
    s = lax.axis_index("s")
    rbase = s * ROWS_PER_TILE
    bufs = (rows0, rows1, rows2, rows3)
    sems = (sem0, sem1, sem2, sem3)
    ssems = (ssem0, ssem1, ssem2, ssem3)

    rslice = pl.ds(rbase, ROWS_PER_TILE)
    pltpu.sync_copy(z_h.at[rslice], acc_sh.at[rslice])
    pltpu.sync_copy(zv_h.at[rslice], cnt_sh.at[rslice])
    pltpu.sync_copy(ones_h, ones_v)
    plsc.subcore_barrier()

    def run_edges(x_src, src2_h, dst2_h):
      irow = s * CHUNKS_PER_TILE

      def body(g, carry):
        pltpu.sync_copy(src2_h.at[pl.ds(irow + g * SUPER, SUPER)], idx_s)
        pltpu.sync_copy(dst2_h.at[pl.ds(irow + g * SUPER, SUPER)], idx_d)
        cnt_descs = []
        NB = len(bufs)
        descs = [pltpu.async_copy(x_src.at[idx_s.at[j]], bufs[j], sems[j])
                 for j in range(NB - 1)]
        descs.append(None)
        scat = [None] * NB
        for j in range(SUPER):
          b = j % NB
          descs[b].wait()
          nxt = j + NB - 1
          if nxt < SUPER:
            nb = nxt % NB
            if scat[nb] is not None:
              scat[nb].wait()
              scat[nb] = None
            descs[nb] = pltpu.async_copy(
                x_src.at[idx_s.at[nxt]], bufs[nb], sems[nb])
          scat[b] = pltpu.async_copy(bufs[b], acc_sh.at[idx_d.at[j]],
                                     ssems[b], add=True)
          cnt_descs.append(
              pltpu.async_copy(ones_v, cnt_sh.at[idx_d.at[j]], semc,
                               add=True))
        for dsc in scat:
          if dsc is not None:
            dsc.wait()
        for dsc in cnt_descs:
          dsc.wait()
        return carry

      lax.fori_loop(0, SUPERS_PER_TILE, body, 0)

    @pl.when(c == 0)
    def _():
      run_edges(x_u, s_ui, d_ui)

    @pl.when(c == 1)
    def _():
      run_edges(x_i, s_iu, d_iu)

    plsc.subcore_barrier()

    def writeout(agg_out, cnt_out):
      pltpu.sync_copy(acc_sh.at[rslice], agg_out.at[rslice])
      pltpu.sync_copy(cnt_sh.at[rslice], cnt_out.at[rslice])

    @pl.when(c == 0)
    def _():
      writeout(agg_i_out, cnt_i_out)

    @pl.when(c == 1)
    def _():
      writeout(agg_u_out, cnt_u_out)

  return k(x_user, x_item, src_ui, dst_ui, src_iu, dst_iu,
           ones_vec, z_vec, z_rows)


def _tc_post_one(agg, cnt, x_dst, W_l, b_l, W_r, alpha, ln_w, ln_b):
  def body(agg_r, cnt_r, x_r, wl_r, bl_r, wr_r, alpha_r, lnw_r, lnb_r,
           out_r):
    a = alpha_r[0, 0]
    inv_n = 1.0 / (N_NODE * D)
    c = jnp.maximum(cnt_r[...][:N_NODE], 1.0)
    agg_m = agg_r[...][:N_NODE] / c
    h = lax.dot_general(agg_m, wl_r[...], (((1,), (1,)), ((), ())),
                        precision=lax.Precision.HIGHEST,
                        preferred_element_type=jnp.float32)
    h = h + bl_r[...]
    h = h + lax.dot_general(x_r[...], wr_r[...], (((1,), (1,)), ((), ())),
                            precision=lax.Precision.HIGHEST,
                            preferred_element_type=jnp.float32)
    p = jnp.where(h >= 0.0, h, a * h)
    m = jnp.sum(p) * inv_n
    xc = p - m
    v = jnp.sum(xc * xc) * inv_n
    out_r[...] = xc * lax.rsqrt(v + 1e-5) * lnw_r[...] + lnb_r[...]

  return pl.pallas_call(
      body,
      out_shape=jax.ShapeDtypeStruct((N_NODE, D), jnp.float32),
  )(agg, cnt, x_dst, W_l, b_l.reshape(1, D), W_r,
    alpha.reshape(1, 1), ln_w.reshape(1, D), ln_b.reshape(1, D))


@jax.jit
def kernel(x_user, x_item, edge_index_ui, edge_index_iu, batch_user,
           batch_item, batch_size, W_l_ui, b_l_ui, W_r_ui, W_l_iu, b_l_iu,
           W_r_iu, alpha, ln_w_user, ln_b_user, ln_w_item, ln_b_item):
  del batch_user, batch_item

  pad = E_PAD - E
  pad_src = jnp.zeros((pad,), jnp.int32)
  pad_dst = jnp.full((pad,), N_NODE, jnp.int32)

  def prep(v, p):
    return jnp.concatenate([v, p]).reshape(IDX_ROWS, CHUNK)

  src_ui = prep(edge_index_ui[0], pad_src)
  dst_ui = prep(edge_index_ui[1], pad_dst)
  src_iu = prep(edge_index_iu[0], pad_src)
  dst_iu = prep(edge_index_iu[1], pad_dst)

  ones_vec = jnp.ones((CHUNK,), jnp.float32)
  z_vec = jnp.zeros((N_ACC,), jnp.float32)
  z_rows = jnp.zeros((N_ACC, D), jnp.float32)

  agg_i, cnt_i, agg_u, cnt_u = _sc_aggregate(
      x_user, x_item, src_ui, dst_ui, src_iu, dst_iu, ones_vec, z_vec, z_rows)

  out_user = _tc_post_one(agg_u, cnt_u.reshape(N_ACC, 1),
                          x_user, W_l_iu, b_l_iu, W_r_iu,
                          alpha, ln_w_user, ln_b_user)
  out_item = _tc_post_one(agg_i, cnt_i.reshape(N_ACC, 1),
                          x_item, W_l_ui, b_l_ui, W_r_ui,
                          alpha, ln_w_item, ln_b_item)
  return out_user, out_item


# speedup vs baseline: 16.7659x; 1.0215x over previous
import functools
import jax
import jax.numpy as jnp
from jax import lax
from jax.experimental import pallas as pl
from jax.experimental.pallas import tpu as pltpu
from jax.experimental.pallas import tpu_sc as plsc

N_NODE = 10000
D = 128
E = 160000

NS = 16
CHUNK = 64
SUPER = 16
CHUNKS_PER_TILE = 160
SUPERS_PER_TILE = CHUNKS_PER_TILE // SUPER
EDGES_PER_TILE = CHUNK * CHUNKS_PER_TILE
E_PAD = EDGES_PER_TILE * NS
IDX_ROWS = NS * CHUNKS_PER_TILE
ROWS_PER_TILE = 640
N_ACC = ROWS_PER_TILE * NS
ROW_LOOPS = ROWS_PER_TILE // CHUNK


def _sc_aggregate(x_user, x_item, src_ui, dst_ui, src_iu, dst_iu,
                  ones_vec, z_vec, z_rows):
  mesh = plsc.VectorSubcoreMesh(core_axis_name="c", subcore_axis_name="s")

  @functools.partial(
      pl.kernel,
      out_type=(
          jax.ShapeDtypeStruct((N_ACC, D), jnp.float32),
          jax.ShapeDtypeStruct((N_ACC,), jnp.float32),
          jax.ShapeDtypeStruct((N_ACC, D), jnp.float32),
          jax.ShapeDtypeStruct((N_ACC,), jnp.float32),
      ),
      mesh=mesh,
      scratch_types=[
          pltpu.VMEM_SHARED((N_ACC, D), jnp.float32),
          pltpu.VMEM_SHARED((N_ACC,), jnp.float32),
          pltpu.VMEM((SUPER, CHUNK), jnp.int32),
          pltpu.VMEM((SUPER, CHUNK), jnp.int32),
          pltpu.VMEM((CHUNK, D), jnp.float32),
          pltpu.VMEM((CHUNK, D), jnp.float32),
          pltpu.VMEM((CHUNK, D), jnp.float32),
          pltpu.VMEM((CHUNK, D), jnp.float32),
          pltpu.VMEM((CHUNK,), jnp.float32),
          pltpu.SemaphoreType.DMA,
          pltpu.SemaphoreType.DMA,
          pltpu.SemaphoreType.DMA,
          pltpu.SemaphoreType.DMA,
          pltpu.SemaphoreType.DMA,
          pltpu.SemaphoreType.DMA,
          pltpu.SemaphoreType.DMA,
          pltpu.SemaphoreType.DMA,
          pltpu.SemaphoreType.DMA,
      ],
  )
  def k(x_u, x_i, s_ui, d_ui, s_iu, d_iu, ones_h, zv_h, z_h,
        agg_i_out, cnt_i_out, agg_u_out, cnt_u_out,
        acc_sh, cnt_sh, idx_s, idx_d, rows0, rows1, rows2, rows3, ones_v,
        sem0, sem1, sem2, sem3, ssem0, ssem1, ssem2, ssem3, semc):
    c = lax.axis_index("c")
    s = lax.axis_index("s")
    rbase = s * ROWS_PER_TILE
    bufs = (rows0, rows1, rows2, rows3)
    sems = (sem0, sem1, sem2, sem3)
    ssems = (ssem0, ssem1, ssem2, ssem3)

    rslice = pl.ds(rbase, ROWS_PER_TILE)
    pltpu.sync_copy(z_h.at[rslice], acc_sh.at[rslice])
    pltpu.sync_copy(zv_h.at[rslice], cnt_sh.at[rslice])
    pltpu.sync_copy(ones_h, ones_v)
    plsc.subcore_barrier()

    def run_edges(x_src, src2_h, dst2_h):
      irow = s * CHUNKS_PER_TILE

      def body(g, carry):
        pltpu.sync_copy(src2_h.at[pl.ds(irow + g * SUPER, SUPER)], idx_s)
        pltpu.sync_copy(dst2_h.at[pl.ds(irow + g * SUPER, SUPER)], idx_d)
        cnt_descs = []
        NB = len(bufs)
        descs = [pltpu.async_copy(x_src.at[idx_s.at[j]], bufs[j], sems[j])
                 for j in range(NB - 1)]
        descs.append(None)
        scat = [None] * NB
        for j in range(SUPER):
          b = j % NB
          descs[b].wait()
          nxt = j + NB - 1
          if nxt < SUPER:
            nb = nxt % NB
            if scat[nb] is not None:
              scat[nb].wait()
              scat[nb] = None
            descs[nb] = pltpu.async_copy(
                x_src.at[idx_s.at[nxt]], bufs[nb], sems[nb])
          scat[b] = pltpu.async_copy(bufs[b], acc_sh.at[idx_d.at[j]],
                                     ssems[b], add=True)
          cnt_descs.append(
              pltpu.async_copy(ones_v, cnt_sh.at[idx_d.at[j]], semc,
                               add=True))
        for dsc in scat:
          if dsc is not None:
            dsc.wait()
        for dsc in cnt_descs:
          dsc.wait()
        return carry

      lax.fori_loop(0, SUPERS_PER_TILE, body, 0)

    @pl.when(c == 0)
    def _():
      run_edges(x_u, s_ui, d_ui)

    @pl.when(c == 1)
    def _():
      run_edges(x_i, s_iu, d_iu)

    plsc.subcore_barrier()

    def writeout(agg_out, cnt_out):
      pltpu.sync_copy(acc_sh.at[rslice], agg_out.at[rslice])
      pltpu.sync_copy(cnt_sh.at[rslice], cnt_out.at[rslice])

    @pl.when(c == 0)
    def _():
      writeout(agg_i_out, cnt_i_out)

    @pl.when(c == 1)
    def _():
      writeout(agg_u_out, cnt_u_out)

  return k(x_user, x_item, src_ui, dst_ui, src_iu, dst_iu,
           ones_vec, z_vec, z_rows)


def _tc_post_one(agg, cnt, x_dst, W_l, b_l, W_r, alpha, ln_w, ln_b):
  def body(agg_r, cnt_r, x_r, wl_r, bl_r, wr_r, alpha_r, lnw_r, lnb_r,
           out_r):
    a = alpha_r[0, 0]
    inv_n = 1.0 / (N_NODE * D)
    c = jnp.maximum(cnt_r[...][:N_NODE], 1.0)
    agg_m = agg_r[...][:N_NODE] / c
    h = lax.dot_general(agg_m, wl_r[...], (((1,), (1,)), ((), ())),
                        precision=lax.Precision.HIGHEST,
                        preferred_element_type=jnp.float32)
    h = h + bl_r[...]
    h = h + lax.dot_general(x_r[...], wr_r[...], (((1,), (1,)), ((), ())),
                            precision=lax.Precision.HIGHEST,
                            preferred_element_type=jnp.float32)
    p = jnp.where(h >= 0.0, h, a * h)
    m = jnp.sum(p) * inv_n
    xc = p - m
    v = jnp.sum(xc * xc) * inv_n
    out_r[...] = xc * lax.rsqrt(v + 1e-5) * lnw_r[...] + lnb_r[...]

  return pl.pallas_call(
      body,
      out_shape=jax.ShapeDtypeStruct((N_NODE, D), jnp.float32),
  )(agg, cnt, x_dst, W_l, b_l.reshape(1, D), W_r,
    alpha.reshape(1, 1), ln_w.reshape(1, D), ln_b.reshape(1, D))


@jax.jit
def kernel(x_user, x_item, edge_index_ui, edge_index_iu, batch_user,
           batch_item, batch_size, W_l_ui, b_l_ui, W_r_ui, W_l_iu, b_l_iu,
           W_r_iu, alpha, ln_w_user, ln_b_user, ln_w_item, ln_b_item):
  del batch_user, batch_item

  pad = E_PAD - E
  pad_src = jnp.zeros((pad,), jnp.int32)
  pad_dst = jnp.full((pad,), N_NODE, jnp.int32)

  def prep(v, p):
    return jnp.concatenate([v, p]).reshape(IDX_ROWS, CHUNK)

  src_ui = prep(edge_index_ui[0], pad_src)
  dst_ui = prep(edge_index_ui[1], pad_dst)
  src_iu = prep(edge_index_iu[0], pad_src)
  dst_iu = prep(edge_index_iu[1], pad_dst)

  ones_vec = jnp.ones((CHUNK,), jnp.float32)
  z_vec = jnp.zeros((N_ACC,), jnp.float32)
  z_rows = jnp.zeros((N_ACC, D), jnp.float32)

  agg_i, cnt_i, agg_u, cnt_u = _sc_aggregate(
      x_user, x_item, src_ui, dst_ui, src_iu, dst_iu, ones_vec, z_vec, z_rows)

  out_user = _tc_post_one(agg_u, cnt_u.reshape(N_ACC, 1),
                          x_user, W_l_iu, b_l_iu, W_r_iu,
                          alpha, ln_w_user, ln_b_user)
  out_item = _tc_post_one(agg_i, cnt_i.reshape(N_ACC, 1),
                          x_item, W_l_ui, b_l_ui, W_r_ui,
                          alpha, ln_w_item, ln_b_item)
  return out_user, out_item
